# baseline (speedup 1.0000x reference)
import functools

import jax
import jax.numpy as jnp
from jax import lax
from jax.experimental import pallas as pl
from jax.experimental.pallas import tpu as pltpu
from jax.experimental.pallas import tpu_sc as plsc

_V = 1000001
_D = 300
_DMAIN = 256
_DTAIL = _D - _DMAIN
_NOUT = 4
_B = 4096
_SEQ = 50

_NC, _NS, _L = 2, 16, 16
_NW = _NC * _NS
_BPW = _B // _NW
_PAIRS = _BPW // 2
_NCHUNK = _DMAIN // _L

_TCBLK = 8192
_TCGRID = -(-_V // _TCBLK)
_VPAD = _TCGRID * _TCBLK


def _sc_main(table, idx2d, fcw, fcb, out, idx_v, a0, a1, b0, b1,
             fcw_v, fcb_v, out_v, sA0, sA1, sB0, sB1):
    wid = lax.axis_index("s") * _NC + lax.axis_index("c")

    pltpu.sync_copy(idx2d.at[pl.ds(wid * _BPW, _BPW)], idx_v)
    pltpu.sync_copy(fcw, fcw_v)
    pltpu.sync_copy(fcb, fcb_v)

    lanes = lax.iota(jnp.int32, _L)
    bias = fcb_v[pl.ds(0, _L)]
    zero = jnp.zeros((_L,), jnp.float32)

    viewA = table.at[:, pl.ds(0, 128)]
    viewB = table.at[:, pl.ds(128, 128)]
    bufsA, bufsB = (a0, a1), (b0, b1)
    semsA, semsB = (sA0, sA1), (sB0, sB1)

    def issue(e, r):
        idx = idx_v.at[e]
        pltpu.make_async_copy(
            viewA.at[idx], bufsA[r].at[pl.ds(0, _SEQ)], semsA[r]).start()
        pltpu.make_async_copy(
            viewB.at[idx], bufsB[r].at[pl.ds(0, _SEQ)], semsB[r]).start()

    def wait(e, r):
        idx = idx_v.at[e]
        pltpu.make_async_copy(
            viewA.at[idx], bufsA[r].at[pl.ds(0, _SEQ)], semsA[r]).wait()
        pltpu.make_async_copy(
            viewB.at[idx], bufsB[r].at[pl.ds(0, _SEQ)], semsB[r]).wait()

    def pool_one(e, r, bb):
        bufA, bufB = bufsA[r], bufsB[r]

        def body(l, accs):
            outs = []
            for j in range(_NCHUNK):
                buf = bufA if j < 8 else bufB
                off = (j % 8) * _L
                outs.append(accs[j] + buf[l, pl.ds(off, _L)])
            return tuple(outs)

        accs = lax.fori_loop(0, _SEQ, body, (zero,) * _NCHUNK)

        y = zero
        for o in range(_NOUT):
            part = accs[0] * fcw_v[pl.ds(o * _L, _L)]
            for j in range(1, _NCHUNK):
                part = part + accs[j] * fcw_v[pl.ds((j * _NOUT + o) * _L, _L)]
            for k in (1, 2, 4, 8):
                perm = (lanes + k) & (_L - 1)
                part = part + part.at[perm].get(mode="promise_in_bounds")
            y = jnp.where(lanes == bb * _NOUT + o, part, y)
        return y

    issue(0, 0)
    issue(1, 1)

    def loop(p, carry):
        e = 2 * p
        wait(e, 0)
        y0 = pool_one(e, 0, 0)
        issue(e + 2, 0)
        wait(e + 1, 1)
        y1 = pool_one(e + 1, 1, 1)
        issue(e + 3, 1)
        out_v[pl.ds(2 * _NOUT * p, _L)] = y0 + y1 + bias
        return carry

    lax.fori_loop(0, _PAIRS - 1, loop, 0)
    p = _PAIRS - 1
    wait(2 * p, 0)
    y0 = pool_one(2 * p, 0, 0)
    wait(2 * p + 1, 1)
    y1 = pool_one(2 * p + 1, 1, 1)
    out_v[pl.ds(2 * _NOUT * p, _L)] = y0 + y1 + bias

    pltpu.sync_copy(out_v.at[pl.ds(0, _BPW * _NOUT)],
                    out.at[pl.ds(wid * _BPW * _NOUT, _BPW * _NOUT)])


def _sc_tail(ptail, idx2d, out, idx_v, t0, t1, out_v, s0, s1):
    wid = lax.axis_index("s") * _NC + lax.axis_index("c")
    pltpu.sync_copy(idx2d.at[pl.ds(wid * _BPW, _BPW)], idx_v)

    lanes = lax.iota(jnp.int32, _L)
    bufs = (t0, t1)
    sems = (s0, s1)

    def issue(e, r):
        pltpu.make_async_copy(ptail.at[idx_v.at[e]],
                              bufs[r].at[pl.ds(0, _SEQ)], sems[r]).start()

    def wait(e, r):
        pltpu.make_async_copy(ptail.at[idx_v.at[e]],
                              bufs[r].at[pl.ds(0, _SEQ)], sems[r]).wait()

    def pool_one(r):
        buf = bufs[r]

        def body(l, acc):
            return acc + buf[l, pl.ds(0, _L)]

        return lax.fori_loop(0, _SEQ, body, jnp.zeros((_L,), jnp.float32))

    issue(0, 0)
    issue(1, 1)

    def loop(p, carry):
        e = 2 * p
        wait(e, 0)
        y0 = pool_one(0)
        issue(e + 2, 0)
        wait(e + 1, 1)
        y1 = pool_one(1)
        issue(e + 3, 1)
        y1r = y1.at[(lanes + 12) & (_L - 1)].get(mode="promise_in_bounds")
        out_v[pl.ds(2 * _NOUT * p, _L)] = jnp.where(lanes < _NOUT, y0, y1r)
        return carry

    lax.fori_loop(0, _PAIRS - 1, loop, 0)
    p = _PAIRS - 1
    wait(2 * p, 0)
    y0 = pool_one(0)
    wait(2 * p + 1, 1)
    y1 = pool_one(1)
    y1r = y1.at[(lanes + 12) & (_L - 1)].get(mode="promise_in_bounds")
    out_v[pl.ds(2 * _NOUT * p, _L)] = jnp.where(lanes < _NOUT, y0, y1r)

    pltpu.sync_copy(out_v.at[pl.ds(0, _BPW * _NOUT)],
                    out.at[pl.ds(wid * _BPW * _NOUT, _BPW * _NOUT)])


def _tc_proj(tail_ref, fcwt_ref, out_ref):
    blk = tail_ref[:, _DMAIN:]
    out_ref[...] = jnp.dot(blk, fcwt_ref[...],
                           preferred_element_type=jnp.float32)


@jax.jit
def kernel(x, weights, fc_w, fc_b):
    chunks = [fc_w[:, j * _L:(j + 1) * _L] * (1.0 / _SEQ)
              for j in range(_NCHUNK)]
    fcw = jnp.stack(chunks).reshape(-1)
    fcb = jnp.tile(fc_b, _L // _NOUT)
    fcwt = jnp.zeros((_DTAIL, 128), jnp.float32)
    fcwt = fcwt.at[:, :_NOUT].set(fc_w[:, _DMAIN:].T * (1.0 / _SEQ))

    mesh = plsc.VectorSubcoreMesh(
        core_axis_name="c", subcore_axis_name="s",
        num_cores=_NC, num_subcores=_NS)

    main = pl.kernel(
        _sc_main,
        out_type=jax.ShapeDtypeStruct((_B * _NOUT,), jnp.float32),
        mesh=mesh,
        compiler_params=pltpu.CompilerParams(use_tc_tiling_on_sc=True),
        scratch_types=[
            pltpu.VMEM((_BPW, _SEQ), jnp.int32),
            pltpu.VMEM((64, 128), jnp.float32),
            pltpu.VMEM((64, 128), jnp.float32),
            pltpu.VMEM((64, 128), jnp.float32),
            pltpu.VMEM((64, 128), jnp.float32),
            pltpu.VMEM((_NCHUNK * _NOUT * _L,), jnp.float32),
            pltpu.VMEM((_L,), jnp.float32),
            pltpu.VMEM((_BPW * _NOUT + _L,), jnp.float32),
            pltpu.SemaphoreType.DMA,
            pltpu.SemaphoreType.DMA,
            pltpu.SemaphoreType.DMA,
            pltpu.SemaphoreType.DMA,
        ],
    )

    proj = pl.pallas_call(
        _tc_proj,
        grid=(_TCGRID,),
        in_specs=[
            pl.BlockSpec((_TCBLK, _D), lambda i: (i, 0)),
            pl.BlockSpec((_DTAIL, 128), lambda i: (0, 0)),
        ],
        out_specs=pl.BlockSpec((_TCBLK, 128), lambda i: (i, 0)),
        out_shape=jax.ShapeDtypeStruct((_VPAD, 128), jnp.float32),
    )

    tail = pl.kernel(
        _sc_tail,
        out_type=jax.ShapeDtypeStruct((_B * _NOUT,), jnp.float32),
        mesh=mesh,
        compiler_params=pltpu.CompilerParams(use_tc_tiling_on_sc=True),
        scratch_types=[
            pltpu.VMEM((_BPW, _SEQ), jnp.int32),
            pltpu.VMEM((64, 128), jnp.float32),
            pltpu.VMEM((64, 128), jnp.float32),
            pltpu.VMEM((_BPW * _NOUT + _L,), jnp.float32),
            pltpu.SemaphoreType.DMA,
            pltpu.SemaphoreType.DMA,
        ],
    )

    out_main = main(weights, x, fcw, fcb)
    ptail = proj(weights, fcwt)
    out_tail = tail(ptail, x)
    return (out_main + out_tail).reshape(_B, _NOUT)


# speedup vs baseline: 1.1608x; 1.1608x over previous
import functools

import jax
import jax.numpy as jnp
from jax import lax
from jax.experimental import pallas as pl
from jax.experimental.pallas import tpu as pltpu
from jax.experimental.pallas import tpu_sc as plsc

_V = 1000001
_D = 300
_DMAIN = 256
_DTAIL = _D - _DMAIN
_NOUT = 4
_B = 4096
_SEQ = 50

_NC, _NS, _L = 2, 16, 16
_NW = _NC * _NS
_BPW = _B // _NW
_PAIRS = _BPW // 2
_NCHUNK = _DMAIN // _L

_TCBLK = 8192
_TCGRID = -(-_V // _TCBLK)
_VPAD = _TCGRID * _TCBLK


def _sc_main(table, idx2d, fcw, fcb, out, idx_v, a0, a1, b0, b1,
             fcw_v, fcb_v, out_v, sA0, sA1, sB0, sB1):
    wid = lax.axis_index("s") * _NC + lax.axis_index("c")

    pltpu.sync_copy(idx2d.at[pl.ds(wid * _BPW, _BPW)], idx_v)
    pltpu.sync_copy(fcw, fcw_v)
    pltpu.sync_copy(fcb, fcb_v)

    lanes = lax.iota(jnp.int32, _L)
    bias = fcb_v[pl.ds(0, _L)]
    zero = jnp.zeros((_L,), jnp.float32)

    viewA = table.at[:, pl.ds(0, 128)]
    viewB = table.at[:, pl.ds(128, 128)]
    bufsA, bufsB = (a0, a1), (b0, b1)
    semsA, semsB = (sA0, sA1), (sB0, sB1)

    def issue(e, r):
        idx = idx_v.at[e]
        pltpu.make_async_copy(
            viewA.at[idx], bufsA[r].at[pl.ds(0, _SEQ)], semsA[r]).start()
        pltpu.make_async_copy(
            viewB.at[idx], bufsB[r].at[pl.ds(0, _SEQ)], semsB[r]).start()

    def wait(e, r):
        idx = idx_v.at[e]
        pltpu.make_async_copy(
            viewA.at[idx], bufsA[r].at[pl.ds(0, _SEQ)], semsA[r]).wait()
        pltpu.make_async_copy(
            viewB.at[idx], bufsB[r].at[pl.ds(0, _SEQ)], semsB[r]).wait()

    def pool_one(e, r, bb):
        bufA, bufB = bufsA[r], bufsB[r]

        def body(l, accs):
            outs = []
            for j in range(_NCHUNK):
                buf = bufA if j < 8 else bufB
                off = (j % 8) * _L
                outs.append(accs[j] + buf[l, pl.ds(off, _L)])
            return tuple(outs)

        accs = lax.fori_loop(0, _SEQ, body, (zero,) * _NCHUNK)

        y = zero
        for o in range(_NOUT):
            part = accs[0] * fcw_v[pl.ds(o * _L, _L)]
            for j in range(1, _NCHUNK):
                part = part + accs[j] * fcw_v[pl.ds((j * _NOUT + o) * _L, _L)]
            for k in (1, 2, 4, 8):
                perm = (lanes + k) & (_L - 1)
                part = part + part.at[perm].get(mode="promise_in_bounds")
            y = jnp.where(lanes == bb * _NOUT + o, part, y)
        return y

    issue(0, 0)
    issue(1, 1)

    def loop(p, carry):
        e = 2 * p
        wait(e, 0)
        y0 = pool_one(e, 0, 0)
        issue(e + 2, 0)
        wait(e + 1, 1)
        y1 = pool_one(e + 1, 1, 1)
        issue(e + 3, 1)
        out_v[pl.ds(2 * _NOUT * p, _L)] = y0 + y1 + bias
        return carry

    lax.fori_loop(0, _PAIRS - 1, loop, 0)
    p = _PAIRS - 1
    wait(2 * p, 0)
    y0 = pool_one(2 * p, 0, 0)
    wait(2 * p + 1, 1)
    y1 = pool_one(2 * p + 1, 1, 1)
    out_v[pl.ds(2 * _NOUT * p, _L)] = y0 + y1 + bias

    pltpu.sync_copy(out_v.at[pl.ds(0, _BPW * _NOUT)],
                    out.at[pl.ds(wid * _BPW * _NOUT, _BPW * _NOUT)])


def _sc_tail(ptail, idx2d, out, idx_v, t0, t1, out_v, s0, s1):
    wid = lax.axis_index("s") * _NC + lax.axis_index("c")
    pltpu.sync_copy(idx2d.at[pl.ds(wid * _BPW, _BPW)], idx_v)

    lanes = lax.iota(jnp.int32, _L)
    bufs = (t0, t1)
    sems = (s0, s1)

    def issue(e, r):
        pltpu.make_async_copy(ptail.at[idx_v.at[e]],
                              bufs[r].at[pl.ds(0, _SEQ)], sems[r]).start()

    def wait(e, r):
        pltpu.make_async_copy(ptail.at[idx_v.at[e]],
                              bufs[r].at[pl.ds(0, _SEQ)], sems[r]).wait()

    def pool_one(r):
        buf = bufs[r]

        def body(l, acc):
            return acc + buf[l, pl.ds(0, _L)]

        return lax.fori_loop(0, _SEQ, body, jnp.zeros((_L,), jnp.float32))

    issue(0, 0)
    issue(1, 1)

    def loop(p, carry):
        e = 2 * p
        wait(e, 0)
        y0 = pool_one(0)
        issue(e + 2, 0)
        wait(e + 1, 1)
        y1 = pool_one(1)
        issue(e + 3, 1)
        y1r = y1.at[(lanes + 12) & (_L - 1)].get(mode="promise_in_bounds")
        out_v[pl.ds(2 * _NOUT * p, _L)] = jnp.where(lanes < _NOUT, y0, y1r)
        return carry

    lax.fori_loop(0, _PAIRS - 1, loop, 0)
    p = _PAIRS - 1
    wait(2 * p, 0)
    y0 = pool_one(0)
    wait(2 * p + 1, 1)
    y1 = pool_one(1)
    y1r = y1.at[(lanes + 12) & (_L - 1)].get(mode="promise_in_bounds")
    out_v[pl.ds(2 * _NOUT * p, _L)] = jnp.where(lanes < _NOUT, y0, y1r)

    pltpu.sync_copy(out_v.at[pl.ds(0, _BPW * _NOUT)],
                    out.at[pl.ds(wid * _BPW * _NOUT, _BPW * _NOUT)])


def _tc_proj(tail_ref, fcwt_ref, out_ref):
    blk = tail_ref[...]
    col = lax.broadcasted_iota(jnp.int32, blk.shape, 1)
    blk = jnp.where(col < _DTAIL, blk, 0.0)
    out_ref[...] = jnp.dot(blk, fcwt_ref[...],
                           preferred_element_type=jnp.float32)


@jax.jit
def kernel(x, weights, fc_w, fc_b):
    chunks = [fc_w[:, j * _L:(j + 1) * _L] * (1.0 / _SEQ)
              for j in range(_NCHUNK)]
    fcw = jnp.stack(chunks).reshape(-1)
    fcb = jnp.tile(fc_b, _L // _NOUT)
    fcwt = jnp.zeros((128, 128), jnp.float32)
    fcwt = fcwt.at[:_DTAIL, :_NOUT].set(fc_w[:, _DMAIN:].T * (1.0 / _SEQ))

    mesh = plsc.VectorSubcoreMesh(
        core_axis_name="c", subcore_axis_name="s",
        num_cores=_NC, num_subcores=_NS)

    main = pl.kernel(
        _sc_main,
        out_type=jax.ShapeDtypeStruct((_B * _NOUT,), jnp.float32),
        mesh=mesh,
        compiler_params=pltpu.CompilerParams(use_tc_tiling_on_sc=True),
        scratch_types=[
            pltpu.VMEM((_BPW, _SEQ), jnp.int32),
            pltpu.VMEM((64, 128), jnp.float32),
            pltpu.VMEM((64, 128), jnp.float32),
            pltpu.VMEM((64, 128), jnp.float32),
            pltpu.VMEM((64, 128), jnp.float32),
            pltpu.VMEM((_NCHUNK * _NOUT * _L,), jnp.float32),
            pltpu.VMEM((_L,), jnp.float32),
            pltpu.VMEM((_BPW * _NOUT + _L,), jnp.float32),
            pltpu.SemaphoreType.DMA,
            pltpu.SemaphoreType.DMA,
            pltpu.SemaphoreType.DMA,
            pltpu.SemaphoreType.DMA,
        ],
    )

    proj = pl.pallas_call(
        _tc_proj,
        grid=(_TCGRID,),
        in_specs=[
            pl.BlockSpec((_TCBLK, 128), lambda i: (i, _DMAIN // 128)),
            pl.BlockSpec((128, 128), lambda i: (0, 0)),
        ],
        out_specs=pl.BlockSpec((_TCBLK, 128), lambda i: (i, 0)),
        out_shape=jax.ShapeDtypeStruct((_VPAD, 128), jnp.float32),
    )

    tail = pl.kernel(
        _sc_tail,
        out_type=jax.ShapeDtypeStruct((_B * _NOUT,), jnp.float32),
        mesh=mesh,
        compiler_params=pltpu.CompilerParams(use_tc_tiling_on_sc=True),
        scratch_types=[
            pltpu.VMEM((_BPW, _SEQ), jnp.int32),
            pltpu.VMEM((64, 128), jnp.float32),
            pltpu.VMEM((64, 128), jnp.float32),
            pltpu.VMEM((_BPW * _NOUT + _L,), jnp.float32),
            pltpu.SemaphoreType.DMA,
            pltpu.SemaphoreType.DMA,
        ],
    )

    out_main = main(weights, x, fcw, fcb)
    ptail = proj(weights, fcwt)
    out_tail = tail(ptail, x)
    return (out_main + out_tail).reshape(_B, _NOUT)
